# baseline (speedup 1.0000x reference)
import functools

import jax
import jax.numpy as jnp
from jax import lax
from jax.experimental import pallas as pl
from jax.experimental.pallas import tpu as pltpu
from jax.experimental.pallas import tpu_sc as plsc

B = 16384
D = 64
NC = 2
NS = 16
NW = NC * NS
TPW = B // NW
C = 128
NCH = TPW // C
NG = C // 16


def _make_sc_call():
    mesh = plsc.VectorSubcoreMesh(core_axis_name="c", subcore_axis_name="s")

    @functools.partial(
        pl.kernel,
        mesh=mesh,
        out_type=jax.ShapeDtypeStruct((B,), jnp.float32),
        compiler_params=pltpu.CompilerParams(
            needs_layout_passes=False, use_tc_tiling_on_sc=False),
        scratch_types=[
            pltpu.VMEM((C,), jnp.int32),
            pltpu.VMEM((C,), jnp.int32),
            pltpu.VMEM((C,), jnp.int32),
            pltpu.VMEM((C,), jnp.int32),
            pltpu.VMEM((C,), jnp.int32),
            pltpu.VMEM((C,), jnp.int32),
            pltpu.VMEM((C, D), jnp.float32),
            pltpu.VMEM((C, D), jnp.float32),
            pltpu.VMEM((C, D), jnp.float32),
            pltpu.VMEM((C, D), jnp.float32),
            pltpu.VMEM((C, D), jnp.float32),
            pltpu.VMEM((C, D), jnp.float32),
            pltpu.VMEM((256,), jnp.float32),
            pltpu.VMEM((TPW,), jnp.float32),
            pltpu.SemaphoreType.DMA,
        ],
    )
    def sc_kernel(hp_i, rp_i, tp_i, hn_i, rn_i, tn_i, node, edge, out,
                  hp_idx, rp_idx, tp_idx, hn_idx, rn_idx, tn_idx,
                  hp_v, rp_v, tp_v, hn_v, rn_v, tn_v,
                  tr_v, out_v, sem):
        wid = lax.axis_index("s") * NC + lax.axis_index("c")
        iota16 = lax.iota(jnp.int32, 16)

        def side_dist(hv, rv, tv, base):
            for i in range(16):
                r_ = base + i
                parts = []
                for k in range(D // 16):
                    sl = pl.ds(k * 16, 16)
                    parts.append(jnp.abs(hv[r_, sl] + rv[r_, sl] - tv[r_, sl]))
                part = (parts[0] + parts[1]) + (parts[2] + parts[3])
                plsc.store_scatter(tr_v, [iota16 * 16 + i], part)
            rows = [tr_v[pl.ds(j * 16, 16)] for j in range(16)]
            while len(rows) > 1:
                rows = [rows[2 * j] + rows[2 * j + 1]
                        for j in range(len(rows) // 2)]
            return rows[0]

        for c in range(NCH):
            row = wid * NCH + c
            pltpu.sync_copy(hp_i.at[row], hp_idx)
            pltpu.sync_copy(rp_i.at[row], rp_idx)
            pltpu.sync_copy(tp_i.at[row], tp_idx)
            pltpu.sync_copy(hn_i.at[row], hn_idx)
            pltpu.sync_copy(rn_i.at[row], rn_idx)
            pltpu.sync_copy(tn_i.at[row], tn_idx)
            cps = [
                pltpu.async_copy(node.at[hp_idx], hp_v, sem),
                pltpu.async_copy(edge.at[rp_idx], rp_v, sem),
                pltpu.async_copy(node.at[tp_idx], tp_v, sem),
                pltpu.async_copy(node.at[hn_idx], hn_v, sem),
                pltpu.async_copy(edge.at[rn_idx], rn_v, sem),
                pltpu.async_copy(node.at[tn_idx], tn_v, sem),
            ]
            for cp in cps:
                cp.wait()

            def group_body(g, _):
                base = g * 16
                pos_d = side_dist(hp_v, rp_v, tp_v, base)
                neg_d = side_dist(hn_v, rn_v, tn_v, base)
                loss = jnp.maximum(pos_d - neg_d + 1.0, 0.0)
                out_v[pl.ds(c * C + g * 16, 16)] = loss
                return 0

            lax.fori_loop(0, NG, group_body, 0)

        pltpu.sync_copy(out_v, out.at[pl.ds(wid * TPW, TPW)])

    return sc_kernel


def kernel(pos_triplets, neg_triplets, node_em, edge_em):
    pos = pos_triplets.astype(jnp.int32)
    neg = neg_triplets.astype(jnp.int32)
    idx_arrays = [
        pos[:, 0].reshape(B // C, C),
        pos[:, 1].reshape(B // C, C),
        pos[:, 2].reshape(B // C, C),
        neg[:, 0].reshape(B // C, C),
        neg[:, 1].reshape(B // C, C),
        neg[:, 2].reshape(B // C, C),
    ]
    sc = _make_sc_call()
    return sc(*idx_arrays, node_em, edge_em)
